# Initial kernel scaffold; baseline (speedup 1.0000x reference)
import jax
import jax.numpy as jnp
from jax.experimental import pallas as pl


def kernel(x, points_xyz, rgb, noise):
    raise NotImplementedError("write your pallas kernel here")


# speedup vs baseline: 2.4001x; 2.4001x over previous
import jax
import jax.numpy as jnp
from jax import lax
from jax.experimental import pallas as pl
from jax.experimental.pallas import tpu as pltpu
from jax.experimental.pallas import tpu_sc as plsc

N, L, D = 8, 4096, 384
LEN_KEEP = 1024
LEN_MASK = L - LEN_KEEP
NC, NS, LANES = 2, 16, 16
NV = L // LANES
ROWS_PER_SC = N // NC
TPR = NS // ROWS_PER_SC
SEG = L // TPR
CHUNK = 128
XC = (ROWS_PER_SC * LEN_KEEP) // NS // CHUNK


def _body(noise_hbm, x2d_hbm, pts_hbm, rgb_hbm,
          xg_hbm, ptsk_hbm, ptsm_hbm, rgbk_hbm, rgbm_hbm,
          restore_hbm, mask_hbm,
          noise_v, ka, kb, pa, pb, hist, off, restore_v, shufg_v, mask_v,
          idxx_v, bufx_v, idxp_v, ptsrow_v, rgbrow_v, bufp_v, bufr_v,
          shuf_sh, sem):
    c = lax.axis_index("c")
    s = lax.axis_index("s")
    lane = lax.iota(jnp.int32, 16)

    prow = c * ROWS_PER_SC + s // TPR
    pltpu.sync_copy(pts_hbm.at[prow], ptsrow_v)
    pltpu.sync_copy(rgb_hbm.at[prow], rgbrow_v)

    @pl.when(s < ROWS_PER_SC)
    def _sort():
        row = c * ROWS_PER_SC + s
        pltpu.sync_copy(noise_hbm.at[row], noise_v)

        def p0(v, carry):
            f = noise_v[pl.ds(v * LANES, LANES)]
            k = lax.bitcast_convert_type(f, jnp.int32)
            e = v * LANES + lane
            slot = ((e & (NV - 1)) << 4) | (e >> 8)
            plsc.store_scatter(ka, [slot], k)
            plsc.store_scatter(pa, [slot], e)
            return carry
        lax.fori_loop(0, NV, p0, 0)

        zeros16 = jnp.zeros((LANES,), jnp.int32)
        ones16 = jnp.ones((LANES,), jnp.int32)
        bufs = [(ka, pa), (kb, pb)]
        for p in range(4):
            shift = 8 * p
            src_k, src_p = bufs[p % 2]
            dst_k, dst_p = bufs[(p + 1) % 2]
            last = p == 3

            def pz(h, carry):
                hist[pl.ds(h * LANES, LANES)] = zeros16
                return carry
            lax.fori_loop(0, NV, pz, 0)

            def ph(v, carry):
                k = src_k[pl.ds(v * LANES, LANES)]
                d = (k >> shift) & 0xFF
                plsc.addupdate_scatter(hist, [(d << 4) | lane], ones16)
                return carry
            lax.fori_loop(0, NV, ph, 0)

            def ps(h, carry):
                hv = hist[pl.ds(h * LANES, LANES)]
                inc = plsc.cumsum(hv)
                off[pl.ds(h * LANES, LANES)] = inc - hv + carry
                return carry + jnp.sum(hv)
            lax.fori_loop(0, NV, ps, jnp.int32(0))

            def pc(v, carry):
                k = src_k[pl.ds(v * LANES, LANES)]
                pay = src_p[pl.ds(v * LANES, LANES)]
                d = (k >> shift) & 0xFF
                hidx = (d << 4) | lane
                pos = plsc.load_gather(off, [hidx])
                plsc.store_scatter(off, [hidx], pos + 1)
                if not last:
                    slot = ((pos & (NV - 1)) << 4) | (pos >> 8)
                    plsc.store_scatter(dst_k, [slot], k)
                    plsc.store_scatter(dst_p, [slot], pay)
                else:
                    plsc.store_scatter(restore_v, [pay], pos)
                    plsc.store_scatter(shufg_v, [pos], pay + row * L)
                return carry
            lax.fori_loop(0, NV, pc, 0)

        def pm(v, carry):
            r = restore_v[pl.ds(v * LANES, LANES)]
            mask_v[pl.ds(v * LANES, LANES)] = jnp.where(
                r >= LEN_KEEP, jnp.float32(1.0), jnp.float32(0.0))
            return carry
        lax.fori_loop(0, NV, pm, 0)

        pltpu.sync_copy(shufg_v, shuf_sh.at[s])
        pltpu.sync_copy(restore_v, restore_hbm.at[row])
        pltpu.sync_copy(mask_v, mask_hbm.at[row])

    plsc.subcore_barrier()

    for j in range(XC):
        flat = s * (XC * CHUNK) + j * CHUNK
        r = flat // LEN_KEEP
        pos = flat % LEN_KEEP
        pltpu.sync_copy(shuf_sh.at[r, pl.ds(pos, CHUNK)], idxx_v.at[j])
    for j in range(XC):
        flat = s * (XC * CHUNK) + j * CHUNK
        pltpu.async_copy(x2d_hbm.at[idxx_v.at[j]], bufx_v, sem).wait()
        pltpu.sync_copy(
            bufx_v,
            xg_hbm.at[pl.ds(c * (ROWS_PER_SC * LEN_KEEP) + flat, CHUNK)])

    seg = s % TPR
    pltpu.sync_copy(shuf_sh.at[s // TPR, pl.ds(seg * SEG, SEG)], idxp_v)

    def pg(i, carry):
        ids3 = (idxp_v[pl.ds(i * LANES, LANES)] & (L - 1)) * 3
        orow3 = (i * LANES + lane) * 3
        for col in range(3):
            pv = plsc.load_gather(ptsrow_v, [ids3 + col])
            plsc.store_scatter(bufp_v, [orow3 + col], pv)
            rv = plsc.load_gather(rgbrow_v, [ids3 + col])
            plsc.store_scatter(bufr_v, [orow3 + col], rv)
        return carry
    lax.fori_loop(0, SEG // LANES, pg, 0)

    @pl.when(seg == 0)
    def _keep_out():
        base = prow * LEN_KEEP * 3
        pltpu.sync_copy(bufp_v, ptsk_hbm.at[pl.ds(base, SEG * 3)])
        pltpu.sync_copy(bufr_v, rgbk_hbm.at[pl.ds(base, SEG * 3)])

    @pl.when(seg != 0)
    def _mask_out():
        base = (prow * LEN_MASK + (seg - 1) * SEG) * 3
        pltpu.sync_copy(bufp_v, ptsm_hbm.at[pl.ds(base, SEG * 3)])
        pltpu.sync_copy(bufr_v, rgbm_hbm.at[pl.ds(base, SEG * 3)])


@jax.jit
def _masker(noise, x2d, pts, rgb):
    f = pl.kernel(
        _body,
        out_type=[
            jax.ShapeDtypeStruct((N * LEN_KEEP, D), jnp.float32),
            jax.ShapeDtypeStruct((N * LEN_KEEP * 3,), jnp.float32),
            jax.ShapeDtypeStruct((N * LEN_MASK * 3,), jnp.float32),
            jax.ShapeDtypeStruct((N * LEN_KEEP * 3,), jnp.float32),
            jax.ShapeDtypeStruct((N * LEN_MASK * 3,), jnp.float32),
            jax.ShapeDtypeStruct((N, L), jnp.int32),
            jax.ShapeDtypeStruct((N, L), jnp.float32),
        ],
        mesh=plsc.VectorSubcoreMesh(
            core_axis_name="c", subcore_axis_name="s",
            num_cores=NC, num_subcores=NS),
        compiler_params=pltpu.CompilerParams(needs_layout_passes=False),
        scratch_types=[
            pltpu.VMEM((L,), jnp.float32),
            pltpu.VMEM((L,), jnp.int32),
            pltpu.VMEM((L,), jnp.int32),
            pltpu.VMEM((L,), jnp.int32),
            pltpu.VMEM((L,), jnp.int32),
            pltpu.VMEM((L,), jnp.int32),
            pltpu.VMEM((L,), jnp.int32),
            pltpu.VMEM((L,), jnp.int32),
            pltpu.VMEM((L,), jnp.int32),
            pltpu.VMEM((L,), jnp.float32),
            pltpu.VMEM((XC, CHUNK), jnp.int32),
            pltpu.VMEM((CHUNK, D), jnp.float32),
            pltpu.VMEM((SEG,), jnp.int32),
            pltpu.VMEM((L * 3,), jnp.float32),
            pltpu.VMEM((L * 3,), jnp.float32),
            pltpu.VMEM((SEG * 3,), jnp.float32),
            pltpu.VMEM((SEG * 3,), jnp.float32),
            pltpu.VMEM_SHARED((ROWS_PER_SC, L), jnp.int32),
            pltpu.SemaphoreType.DMA,
        ],
    )
    return f(noise, x2d, pts, rgb)


def kernel(x, points_xyz, rgb, noise):
    x2d = x.reshape(N * L, D)
    xg, ptsk, ptsm, rgbk, rgbm, ids_restore, mask = _masker(
        noise, x2d, points_xyz.reshape(N, L * 3), rgb.reshape(N, L * 3))
    return (xg.reshape(N, LEN_KEEP, D), mask, ids_restore,
            ptsk.reshape(N, LEN_KEEP, 3), ptsm.reshape(N, LEN_MASK, 3),
            rgbk.reshape(N, LEN_KEEP, 3), rgbm.reshape(N, LEN_MASK, 3))


# speedup vs baseline: 3.8043x; 1.5850x over previous
import jax
import jax.numpy as jnp
from jax import lax
from jax.experimental import pallas as pl
from jax.experimental.pallas import tpu as pltpu
from jax.experimental.pallas import tpu_sc as plsc

N, L, D = 8, 4096, 384
LEN_KEEP = 1024
LEN_MASK = L - LEN_KEEP
NC, NS, LANES = 2, 16, 16
NV = L // LANES
ROWS_PER_SC = N // NC
TPR = NS // ROWS_PER_SC
SEG = L // TPR
KSEG = LEN_KEEP // TPR
CHUNK = 128
XC = KSEG // CHUNK
PRGW = 8


KEEPBLK = 6 * N * LEN_KEEP


def _body(noise_hbm, x2d_hbm, prg_hbm,
          xg_hbm, planes_hbm, restore_hbm, mask_hbm,
          ka, kb, pa, pb, hist, off, restore_v, shufg_v, mask_v,
          idxx_v, bufx_v, idxp_v, prgrow_v, b0, b1, b2, b3, b4, b5,
          shuf_sh, sem):
    c = lax.axis_index("c")
    s = lax.axis_index("s")
    lane = lax.iota(jnp.int32, 16)
    zeros16 = jnp.zeros((LANES,), jnp.int32)
    ones16 = jnp.ones((LANES,), jnp.int32)

    prow = c * ROWS_PER_SC + s // TPR
    pltpu.sync_copy(prg_hbm.at[prow], prgrow_v)

    @pl.when(s < ROWS_PER_SC)
    def _sort():
        row = c * ROWS_PER_SC + s
        pltpu.sync_copy(noise_hbm.at[row], mask_v)

        def p0(v, carry):
            f = mask_v[pl.ds(v * LANES, LANES)]
            k = lax.bitcast_convert_type(f, jnp.int32)
            e = v * LANES + lane
            slot = ((e & (NV - 1)) << 4) | (e >> 8)
            plsc.store_scatter(ka, [slot], k)
            plsc.store_scatter(pa, [slot], e)
            return carry
        lax.fori_loop(0, NV, p0, 0)

        bufs = [(ka, pa), (kb, pb)]
        for p in range(4):
            shift = 8 * p
            src_k, src_p = bufs[p % 2]
            dst_k, dst_p = bufs[(p + 1) % 2]
            last = p == 3

            def pz(h, carry):
                hist[pl.ds(h * LANES, LANES)] = zeros16
                return carry
            lax.fori_loop(0, NV, pz, 0)

            def ph(v, carry):
                k = src_k[pl.ds(v * LANES, LANES)]
                d = (k >> shift) & 0xFF
                plsc.addupdate_scatter(hist, [(d << 4) | lane], ones16)
                return carry
            lax.fori_loop(0, NV, ph, 0)

            def ps(h, carry):
                hv = hist[pl.ds(h * LANES, LANES)]
                inc = plsc.cumsum(hv)
                off[pl.ds(h * LANES, LANES)] = inc - hv + carry
                return carry + jnp.sum(hv)
            lax.fori_loop(0, NV, ps, jnp.int32(0))

            def pc(v, carry):
                k = src_k[pl.ds(v * LANES, LANES)]
                pay = src_p[pl.ds(v * LANES, LANES)]
                d = (k >> shift) & 0xFF
                hidx = (d << 4) | lane
                pos = plsc.load_gather(off, [hidx])
                plsc.store_scatter(off, [hidx], pos + 1)
                if not last:
                    slot = ((pos & (NV - 1)) << 4) | (pos >> 8)
                    plsc.store_scatter(dst_k, [slot], k)
                    plsc.store_scatter(dst_p, [slot], pay)
                else:
                    plsc.store_scatter(restore_v, [pay], pos)
                    plsc.store_scatter(shufg_v, [pos], pay + row * L)
                return carry
            lax.fori_loop(0, NV, pc, 0)

        def pm(v, carry):
            r = restore_v[pl.ds(v * LANES, LANES)]
            mask_v[pl.ds(v * LANES, LANES)] = jnp.where(
                r >= LEN_KEEP, jnp.float32(1.0), jnp.float32(0.0))
            return carry
        lax.fori_loop(0, NV, pm, 0)

        pltpu.sync_copy(shufg_v, shuf_sh.at[s])
        pltpu.sync_copy(restore_v, restore_hbm.at[row])
        pltpu.sync_copy(mask_v, mask_hbm.at[row])

    plsc.subcore_barrier()

    r = s // TPR
    seg = s % TPR
    for j in range(XC):
        pltpu.sync_copy(shuf_sh.at[r, pl.ds(seg * KSEG + j * CHUNK, CHUNK)],
                        idxx_v.at[j])
    for j in range(XC):
        pltpu.async_copy(x2d_hbm.at[idxx_v.at[j]], bufx_v, sem).wait()
        outbase = c * (ROWS_PER_SC * LEN_KEEP) + s * KSEG + j * CHUNK
        pltpu.sync_copy(bufx_v, xg_hbm.at[pl.ds(outbase, CHUNK)])

    pltpu.sync_copy(shuf_sh.at[r, pl.ds(seg * SEG, SEG)], idxp_v)
    planes = [b0, b1, b2, b3, b4, b5]

    def pg(i, carry):
        ids8 = (idxp_v[pl.ds(i * LANES, LANES)] & (L - 1)) * PRGW
        for col in range(6):
            planes[col][pl.ds(i * LANES, LANES)] = plsc.load_gather(
                prgrow_v, [ids8 + col])
        return carry
    lax.fori_loop(0, SEG // LANES, pg, 0)

    @pl.when(seg == 0)
    def _keep_out():
        for col in range(6):
            base = col * (N * LEN_KEEP) + prow * LEN_KEEP
            pltpu.sync_copy(planes[col], planes_hbm.at[pl.ds(base, SEG)])

    @pl.when(seg != 0)
    def _mask_out():
        for col in range(6):
            base = (KEEPBLK + col * (N * LEN_MASK)
                    + prow * LEN_MASK + (seg - 1) * SEG)
            pltpu.sync_copy(planes[col], planes_hbm.at[pl.ds(base, SEG)])


@jax.jit
def _masker(noise, x2d, prg):
    i32 = jnp.int32
    f32 = jnp.float32
    f = pl.kernel(
        _body,
        out_type=[
            jax.ShapeDtypeStruct((N * LEN_KEEP, D), f32),
            jax.ShapeDtypeStruct((6 * N * L,), f32),
            jax.ShapeDtypeStruct((N, L), i32),
            jax.ShapeDtypeStruct((N, L), f32),
        ],
        mesh=plsc.VectorSubcoreMesh(
            core_axis_name="c", subcore_axis_name="s",
            num_cores=NC, num_subcores=NS),
        compiler_params=pltpu.CompilerParams(needs_layout_passes=False),
        scratch_types=[
            pltpu.VMEM((L,), i32),
            pltpu.VMEM((L,), i32),
            pltpu.VMEM((L,), i32),
            pltpu.VMEM((L,), i32),
            pltpu.VMEM((L,), i32),
            pltpu.VMEM((L,), i32),
            pltpu.VMEM((L,), i32),
            pltpu.VMEM((L,), i32),
            pltpu.VMEM((L,), f32),
            pltpu.VMEM((XC, CHUNK), i32),
            pltpu.VMEM((CHUNK, D), f32),
            pltpu.VMEM((SEG,), i32),
            pltpu.VMEM((L * PRGW,), f32),
            pltpu.VMEM((SEG,), f32),
            pltpu.VMEM((SEG,), f32),
            pltpu.VMEM((SEG,), f32),
            pltpu.VMEM((SEG,), f32),
            pltpu.VMEM((SEG,), f32),
            pltpu.VMEM((SEG,), f32),
            pltpu.VMEM_SHARED((ROWS_PER_SC, L), i32),
            pltpu.SemaphoreType.DMA,
        ],
    )
    return f(noise, x2d, prg)


def kernel(x, points_xyz, rgb, noise):
    x2d = x.reshape(N * L, D)
    prg = jnp.concatenate(
        [points_xyz, rgb, jnp.zeros((N, L, PRGW - 6), jnp.float32)],
        axis=-1).reshape(N, L * PRGW)
    xg, planes, ids_restore, mask = _masker(noise, x2d, prg)
    nk, nm = N * LEN_KEEP, N * LEN_MASK
    kp = planes[:6 * nk].reshape(6, N, LEN_KEEP)
    mp = planes[6 * nk:].reshape(6, N, LEN_MASK)
    pk = jnp.stack([kp[0], kp[1], kp[2]], axis=-1)
    rk = jnp.stack([kp[3], kp[4], kp[5]], axis=-1)
    pm = jnp.stack([mp[0], mp[1], mp[2]], axis=-1)
    rm = jnp.stack([mp[3], mp[4], mp[5]], axis=-1)
    return (xg.reshape(N, LEN_KEEP, D), mask, ids_restore, pk, pm, rk, rm)


# speedup vs baseline: 4.7750x; 1.2552x over previous
import jax
import jax.numpy as jnp
from jax import lax
from jax.experimental import pallas as pl
from jax.experimental.pallas import tpu as pltpu
from jax.experimental.pallas import tpu_sc as plsc

N, L, D = 8, 4096, 384
LEN_KEEP = 1024
LEN_MASK = L - LEN_KEEP
NC, NS, LANES = 2, 16, 16
NV = L // LANES
ROWS_PER_SC = N // NC
TPR = NS // ROWS_PER_SC
SEG = L // TPR
KSEG = LEN_KEEP // TPR
CHUNK = 128
XC = KSEG // CHUNK
PRGW = 8
KEEPBLK = 6 * N * LEN_KEEP


def _sort_body(noise_hbm, restore_hbm, mask_hbm, shufg_hbm,
               noise_v, ka, kb, pa, pb, hist, off, restore_v, shufg_v,
               mask_v):
    c = lax.axis_index("c")
    s = lax.axis_index("s")
    lane = lax.iota(jnp.int32, 16)
    zeros16 = jnp.zeros((LANES,), jnp.int32)
    ones16 = jnp.ones((LANES,), jnp.int32)

    @pl.when(s < ROWS_PER_SC)
    def _sort():
        row = c * ROWS_PER_SC + s
        pltpu.sync_copy(noise_hbm.at[row], noise_v)

        def p0(v, carry):
            f = noise_v[pl.ds(v * LANES, LANES)]
            k = lax.bitcast_convert_type(f, jnp.int32)
            e = v * LANES + lane
            slot = ((e & (NV - 1)) << 4) | (e >> 8)
            plsc.store_scatter(ka, [slot], k)
            plsc.store_scatter(pa, [slot], e)
            return carry
        lax.fori_loop(0, NV, p0, 0)

        bufs = [(ka, pa), (kb, pb)]
        for p in range(4):
            shift = 8 * p
            src_k, src_p = bufs[p % 2]
            dst_k, dst_p = bufs[(p + 1) % 2]
            last = p == 3

            def pz(h, carry):
                hist[pl.ds(h * LANES, LANES)] = zeros16
                return carry
            lax.fori_loop(0, NV, pz, 0)

            def ph(v, carry):
                k = src_k[pl.ds(v * LANES, LANES)]
                d = (k >> shift) & 0xFF
                plsc.addupdate_scatter(hist, [(d << 4) | lane], ones16)
                return carry
            lax.fori_loop(0, NV, ph, 0)

            def ps(h, carry):
                hv = hist[pl.ds(h * LANES, LANES)]
                inc = plsc.cumsum(hv)
                off[pl.ds(h * LANES, LANES)] = inc - hv + carry
                return carry + jnp.sum(hv)
            lax.fori_loop(0, NV, ps, jnp.int32(0))

            def pc(v, carry):
                k = src_k[pl.ds(v * LANES, LANES)]
                pay = src_p[pl.ds(v * LANES, LANES)]
                d = (k >> shift) & 0xFF
                hidx = (d << 4) | lane
                pos = plsc.load_gather(off, [hidx])
                plsc.store_scatter(off, [hidx], pos + 1)
                if not last:
                    slot = ((pos & (NV - 1)) << 4) | (pos >> 8)
                    plsc.store_scatter(dst_k, [slot], k)
                    plsc.store_scatter(dst_p, [slot], pay)
                else:
                    plsc.store_scatter(restore_v, [pay], pos)
                    plsc.store_scatter(shufg_v, [pos], pay + row * L)
                return carry
            lax.fori_loop(0, NV, pc, 0)

        def pm(v, carry):
            r = restore_v[pl.ds(v * LANES, LANES)]
            mask_v[pl.ds(v * LANES, LANES)] = jnp.where(
                r >= LEN_KEEP, jnp.float32(1.0), jnp.float32(0.0))
            return carry
        lax.fori_loop(0, NV, pm, 0)

        pltpu.sync_copy(restore_v, restore_hbm.at[row])
        pltpu.sync_copy(mask_v, mask_hbm.at[row])
        pltpu.sync_copy(shufg_v, shufg_hbm.at[row])


def _gather_body(x2d_hbm, prg_hbm, shufg_hbm,
                 xg_hbm, planes_hbm,
                 idxx_v, bufx_v, idxp_v, prgrow_v, b0, b1, b2, b3, b4, b5,
                 sem):
    c = lax.axis_index("c")
    s = lax.axis_index("s")

    prow = c * ROWS_PER_SC + s // TPR
    seg = s % TPR

    pltpu.sync_copy(prg_hbm.at[prow], prgrow_v)
    for j in range(XC):
        pltpu.sync_copy(shufg_hbm.at[prow, pl.ds(seg * KSEG + j * CHUNK,
                                                 CHUNK)],
                        idxx_v.at[j])
    pltpu.sync_copy(shufg_hbm.at[prow, pl.ds(seg * SEG, SEG)], idxp_v)

    for j in range(XC):
        pltpu.async_copy(x2d_hbm.at[idxx_v.at[j]], bufx_v, sem).wait()
        outbase = c * (ROWS_PER_SC * LEN_KEEP) + s * KSEG + j * CHUNK
        pltpu.sync_copy(bufx_v, xg_hbm.at[pl.ds(outbase, CHUNK)])

    planes = [b0, b1, b2, b3, b4, b5]

    def pg(i, carry):
        ids8 = (idxp_v[pl.ds(i * LANES, LANES)] & (L - 1)) * PRGW
        for col in range(6):
            planes[col][pl.ds(i * LANES, LANES)] = plsc.load_gather(
                prgrow_v, [ids8 + col])
        return carry
    lax.fori_loop(0, SEG // LANES, pg, 0)

    @pl.when(seg == 0)
    def _keep_out():
        for col in range(6):
            base = col * (N * LEN_KEEP) + prow * LEN_KEEP
            pltpu.sync_copy(planes[col], planes_hbm.at[pl.ds(base, SEG)])

    @pl.when(seg != 0)
    def _mask_out():
        for col in range(6):
            base = (KEEPBLK + col * (N * LEN_MASK)
                    + prow * LEN_MASK + (seg - 1) * SEG)
            pltpu.sync_copy(planes[col], planes_hbm.at[pl.ds(base, SEG)])


@jax.jit
def _masker(noise, x2d, prg):
    i32 = jnp.int32
    f32 = jnp.float32
    mesh = plsc.VectorSubcoreMesh(
        core_axis_name="c", subcore_axis_name="s",
        num_cores=NC, num_subcores=NS)
    params = pltpu.CompilerParams(needs_layout_passes=False)

    sortk = pl.kernel(
        _sort_body,
        out_type=[
            jax.ShapeDtypeStruct((N, L), i32),
            jax.ShapeDtypeStruct((N, L), f32),
            jax.ShapeDtypeStruct((N, L), i32),
        ],
        mesh=mesh,
        compiler_params=params,
        scratch_types=[
            pltpu.VMEM((L,), f32),
            pltpu.VMEM((L,), i32),
            pltpu.VMEM((L,), i32),
            pltpu.VMEM((L,), i32),
            pltpu.VMEM((L,), i32),
            pltpu.VMEM((L,), i32),
            pltpu.VMEM((L,), i32),
            pltpu.VMEM((L,), i32),
            pltpu.VMEM((L,), i32),
            pltpu.VMEM((L,), f32),
        ],
    )
    ids_restore, mask, shufg = sortk(noise)

    gatherk = pl.kernel(
        _gather_body,
        out_type=[
            jax.ShapeDtypeStruct((N * LEN_KEEP, D), f32),
            jax.ShapeDtypeStruct((6 * N * L,), f32),
        ],
        mesh=mesh,
        compiler_params=params,
        scratch_types=[
            pltpu.VMEM((XC, CHUNK), i32),
            pltpu.VMEM((CHUNK, D), f32),
            pltpu.VMEM((SEG,), i32),
            pltpu.VMEM((L * PRGW,), f32),
            pltpu.VMEM((SEG,), f32),
            pltpu.VMEM((SEG,), f32),
            pltpu.VMEM((SEG,), f32),
            pltpu.VMEM((SEG,), f32),
            pltpu.VMEM((SEG,), f32),
            pltpu.VMEM((SEG,), f32),
            pltpu.SemaphoreType.DMA,
        ],
    )
    xg, planes = gatherk(x2d, prg, shufg)
    return xg, planes, ids_restore, mask


def kernel(x, points_xyz, rgb, noise):
    x2d = x.reshape(N * L, D)
    prg = jnp.concatenate(
        [points_xyz, rgb, jnp.zeros((N, L, PRGW - 6), jnp.float32)],
        axis=-1).reshape(N, L * PRGW)
    xg, planes, ids_restore, mask = _masker(noise, x2d, prg)
    nk = N * LEN_KEEP
    kp = planes[:6 * nk].reshape(6, N, LEN_KEEP)
    mp = planes[6 * nk:].reshape(6, N, LEN_MASK)
    pk = jnp.stack([kp[0], kp[1], kp[2]], axis=-1)
    rk = jnp.stack([kp[3], kp[4], kp[5]], axis=-1)
    pm = jnp.stack([mp[0], mp[1], mp[2]], axis=-1)
    rm = jnp.stack([mp[3], mp[4], mp[5]], axis=-1)
    return (xg.reshape(N, LEN_KEEP, D), mask, ids_restore, pk, pm, rk, rm)


# speedup vs baseline: 4.7767x; 1.0003x over previous
import jax
import jax.numpy as jnp
from jax import lax
from jax.experimental import pallas as pl
from jax.experimental.pallas import tpu as pltpu
from jax.experimental.pallas import tpu_sc as plsc

N, L, D = 8, 4096, 384
LEN_KEEP = 1024
LEN_MASK = L - LEN_KEEP
NC, NS, LANES = 2, 16, 16
NV = L // LANES
ROWS_PER_SC = N // NC
TPR = NS // ROWS_PER_SC
SEG = L // TPR
KSEG = LEN_KEEP // TPR
CHUNK = 128
XC = KSEG // CHUNK
PRGW = 8
KEEPBLK = 6 * N * LEN_KEEP


def _sort_body(noise_hbm, restore_hbm, mask_hbm, shufg_hbm,
               noise_v, ka, kb, pa, pb, hist, off, restore_v, shufg_v,
               mask_v):
    c = lax.axis_index("c")
    s = lax.axis_index("s")
    lane = lax.iota(jnp.int32, 16)
    zeros16 = jnp.zeros((LANES,), jnp.int32)
    ones16 = jnp.ones((LANES,), jnp.int32)

    @pl.when(s < ROWS_PER_SC)
    def _sort():
        row = c * ROWS_PER_SC + s
        pltpu.sync_copy(noise_hbm.at[row], noise_v)

        def p0(v, carry):
            f = noise_v[pl.ds(v * LANES, LANES)]
            k = lax.bitcast_convert_type(f, jnp.int32)
            e = v * LANES + lane
            slot = ((e & (NV - 1)) << 4) | (e >> 8)
            plsc.store_scatter(ka, [slot], k)
            plsc.store_scatter(pa, [slot], e)
            return carry
        lax.fori_loop(0, NV, p0, 0)

        bufs = [(ka, pa), (kb, pb)]
        for p in range(4):
            shift = 8 * p
            src_k, src_p = bufs[p % 2]
            dst_k, dst_p = bufs[(p + 1) % 2]
            last = p == 3

            def pz(h, carry):
                hist[pl.ds(h * LANES, LANES)] = zeros16
                return carry
            lax.fori_loop(0, NV, pz, 0)

            def ph(v, carry):
                for u in range(2):
                    k = src_k[pl.ds((2 * v + u) * LANES, LANES)]
                    d = (k >> shift) & 0xFF
                    plsc.addupdate_scatter(hist, [(d << 4) | lane], ones16)
                return carry
            lax.fori_loop(0, NV // 2, ph, 0)

            def ps(h, carry):
                hv = hist[pl.ds(h * LANES, LANES)]
                inc = plsc.cumsum(hv)
                off[pl.ds(h * LANES, LANES)] = inc - hv + carry
                return carry + jnp.sum(hv)
            lax.fori_loop(0, NV, ps, jnp.int32(0))

            def pc(v, carry):
                for u in range(2):
                    k = src_k[pl.ds((2 * v + u) * LANES, LANES)]
                    pay = src_p[pl.ds((2 * v + u) * LANES, LANES)]
                    d = (k >> shift) & 0xFF
                    hidx = (d << 4) | lane
                    pos = plsc.load_gather(off, [hidx])
                    plsc.store_scatter(off, [hidx], pos + 1)
                    if not last:
                        slot = ((pos & (NV - 1)) << 4) | (pos >> 8)
                        plsc.store_scatter(dst_k, [slot], k)
                        plsc.store_scatter(dst_p, [slot], pay)
                    else:
                        plsc.store_scatter(restore_v, [pay], pos)
                        plsc.store_scatter(shufg_v, [pos], pay + row * L)
                return carry
            lax.fori_loop(0, NV // 2, pc, 0)

        def pm(v, carry):
            r = restore_v[pl.ds(v * LANES, LANES)]
            mask_v[pl.ds(v * LANES, LANES)] = jnp.where(
                r >= LEN_KEEP, jnp.float32(1.0), jnp.float32(0.0))
            return carry
        lax.fori_loop(0, NV, pm, 0)

        pltpu.sync_copy(restore_v, restore_hbm.at[row])
        pltpu.sync_copy(mask_v, mask_hbm.at[row])
        pltpu.sync_copy(shufg_v, shufg_hbm.at[row])


def _gather_body(x2d_hbm, prg_hbm, shufg_hbm,
                 xg_hbm, planes_hbm,
                 idxx_v, bufx_v, idxp_v, prgrow_v, b0, b1, b2, b3, b4, b5,
                 sem):
    c = lax.axis_index("c")
    s = lax.axis_index("s")

    prow = c * ROWS_PER_SC + s // TPR
    seg = s % TPR

    pltpu.sync_copy(prg_hbm.at[prow], prgrow_v)
    for j in range(XC):
        pltpu.sync_copy(shufg_hbm.at[prow, pl.ds(seg * KSEG + j * CHUNK,
                                                 CHUNK)],
                        idxx_v.at[j])
    pltpu.sync_copy(shufg_hbm.at[prow, pl.ds(seg * SEG, SEG)], idxp_v)

    for j in range(XC):
        pltpu.async_copy(x2d_hbm.at[idxx_v.at[j]], bufx_v, sem).wait()
        outbase = c * (ROWS_PER_SC * LEN_KEEP) + s * KSEG + j * CHUNK
        pltpu.sync_copy(bufx_v, xg_hbm.at[pl.ds(outbase, CHUNK)])

    planes = [b0, b1, b2, b3, b4, b5]

    def pg(i, carry):
        ids8 = (idxp_v[pl.ds(i * LANES, LANES)] & (L - 1)) * PRGW
        for col in range(6):
            planes[col][pl.ds(i * LANES, LANES)] = plsc.load_gather(
                prgrow_v, [ids8 + col])
        return carry
    lax.fori_loop(0, SEG // LANES, pg, 0)

    @pl.when(seg == 0)
    def _keep_out():
        for col in range(6):
            base = col * (N * LEN_KEEP) + prow * LEN_KEEP
            pltpu.sync_copy(planes[col], planes_hbm.at[pl.ds(base, SEG)])

    @pl.when(seg != 0)
    def _mask_out():
        for col in range(6):
            base = (KEEPBLK + col * (N * LEN_MASK)
                    + prow * LEN_MASK + (seg - 1) * SEG)
            pltpu.sync_copy(planes[col], planes_hbm.at[pl.ds(base, SEG)])


@jax.jit
def _masker(noise, x2d, prg):
    i32 = jnp.int32
    f32 = jnp.float32
    mesh = plsc.VectorSubcoreMesh(
        core_axis_name="c", subcore_axis_name="s",
        num_cores=NC, num_subcores=NS)
    params = pltpu.CompilerParams(needs_layout_passes=False)

    sortk = pl.kernel(
        _sort_body,
        out_type=[
            jax.ShapeDtypeStruct((N, L), i32),
            jax.ShapeDtypeStruct((N, L), f32),
            jax.ShapeDtypeStruct((N, L), i32),
        ],
        mesh=mesh,
        compiler_params=params,
        scratch_types=[
            pltpu.VMEM((L,), f32),
            pltpu.VMEM((L,), i32),
            pltpu.VMEM((L,), i32),
            pltpu.VMEM((L,), i32),
            pltpu.VMEM((L,), i32),
            pltpu.VMEM((L,), i32),
            pltpu.VMEM((L,), i32),
            pltpu.VMEM((L,), i32),
            pltpu.VMEM((L,), i32),
            pltpu.VMEM((L,), f32),
        ],
    )
    ids_restore, mask, shufg = sortk(noise)

    gatherk = pl.kernel(
        _gather_body,
        out_type=[
            jax.ShapeDtypeStruct((N * LEN_KEEP, D), f32),
            jax.ShapeDtypeStruct((6 * N * L,), f32),
        ],
        mesh=mesh,
        compiler_params=params,
        scratch_types=[
            pltpu.VMEM((XC, CHUNK), i32),
            pltpu.VMEM((CHUNK, D), f32),
            pltpu.VMEM((SEG,), i32),
            pltpu.VMEM((L * PRGW,), f32),
            pltpu.VMEM((SEG,), f32),
            pltpu.VMEM((SEG,), f32),
            pltpu.VMEM((SEG,), f32),
            pltpu.VMEM((SEG,), f32),
            pltpu.VMEM((SEG,), f32),
            pltpu.VMEM((SEG,), f32),
            pltpu.SemaphoreType.DMA,
        ],
    )
    xg, planes = gatherk(x2d, prg, shufg)
    return xg, planes, ids_restore, mask


def kernel(x, points_xyz, rgb, noise):
    x2d = x.reshape(N * L, D)
    prg = jnp.concatenate(
        [points_xyz, rgb, jnp.zeros((N, L, PRGW - 6), jnp.float32)],
        axis=-1).reshape(N, L * PRGW)
    xg, planes, ids_restore, mask = _masker(noise, x2d, prg)
    nk = N * LEN_KEEP
    kp = planes[:6 * nk].reshape(6, N, LEN_KEEP)
    mp = planes[6 * nk:].reshape(6, N, LEN_MASK)
    pk = jnp.stack([kp[0], kp[1], kp[2]], axis=-1)
    rk = jnp.stack([kp[3], kp[4], kp[5]], axis=-1)
    pm = jnp.stack([mp[0], mp[1], mp[2]], axis=-1)
    rm = jnp.stack([mp[3], mp[4], mp[5]], axis=-1)
    return (xg.reshape(N, LEN_KEEP, D), mask, ids_restore, pk, pm, rk, rm)


# speedup vs baseline: 4.8427x; 1.0138x over previous
import jax
import jax.numpy as jnp
from jax import lax
from jax.experimental import pallas as pl
from jax.experimental.pallas import tpu as pltpu
from jax.experimental.pallas import tpu_sc as plsc

N, L, D = 8, 4096, 384
LEN_KEEP = 1024
LEN_MASK = L - LEN_KEEP
NC, NS, LANES = 2, 16, 16
NV = L // LANES
ROWS_PER_SC = N // NC
TPR = NS // ROWS_PER_SC
SEG = L // TPR
KSEG = LEN_KEEP // TPR
CHUNK = 64
XC = KSEG // CHUNK
PRGW = 8
KEEPBLK = 6 * N * LEN_KEEP


def _sort_body(noise_hbm, restore_hbm, mask_hbm, shufg_hbm,
               noise_v, ka, kb, pa, pb, hist, off, restore_v, shufg_v,
               mask_v):
    c = lax.axis_index("c")
    s = lax.axis_index("s")
    lane = lax.iota(jnp.int32, 16)
    zeros16 = jnp.zeros((LANES,), jnp.int32)
    ones16 = jnp.ones((LANES,), jnp.int32)

    @pl.when(s < ROWS_PER_SC)
    def _sort():
        row = c * ROWS_PER_SC + s
        pltpu.sync_copy(noise_hbm.at[row], noise_v)

        def p0(v, carry):
            f = noise_v[pl.ds(v * LANES, LANES)]
            k = lax.bitcast_convert_type(f, jnp.int32)
            e = v * LANES + lane
            slot = ((e & (NV - 1)) << 4) | (e >> 8)
            plsc.store_scatter(ka, [slot], k)
            plsc.store_scatter(pa, [slot], e)
            return carry
        lax.fori_loop(0, NV, p0, 0)

        bufs = [(ka, pa), (kb, pb)]
        for p in range(4):
            shift = 8 * p
            src_k, src_p = bufs[p % 2]
            dst_k, dst_p = bufs[(p + 1) % 2]
            last = p == 3

            def pz(h, carry):
                hist[pl.ds(h * LANES, LANES)] = zeros16
                return carry
            lax.fori_loop(0, NV, pz, 0)

            def ph(v, carry):
                for u in range(2):
                    k = src_k[pl.ds((2 * v + u) * LANES, LANES)]
                    d = (k >> shift) & 0xFF
                    plsc.addupdate_scatter(hist, [(d << 4) | lane], ones16)
                return carry
            lax.fori_loop(0, NV // 2, ph, 0)

            def ps(h, carry):
                hv = hist[pl.ds(h * LANES, LANES)]
                inc = plsc.cumsum(hv)
                off[pl.ds(h * LANES, LANES)] = inc - hv + carry
                return carry + jnp.sum(hv)
            lax.fori_loop(0, NV, ps, jnp.int32(0))

            def pc(v, carry):
                for u in range(2):
                    k = src_k[pl.ds((2 * v + u) * LANES, LANES)]
                    pay = src_p[pl.ds((2 * v + u) * LANES, LANES)]
                    d = (k >> shift) & 0xFF
                    hidx = (d << 4) | lane
                    pos = plsc.load_gather(off, [hidx])
                    plsc.store_scatter(off, [hidx], pos + 1)
                    if not last:
                        slot = ((pos & (NV - 1)) << 4) | (pos >> 8)
                        plsc.store_scatter(dst_k, [slot], k)
                        plsc.store_scatter(dst_p, [slot], pay)
                    else:
                        plsc.store_scatter(restore_v, [pay], pos)
                        plsc.store_scatter(shufg_v, [pos], pay + row * L)
                return carry
            lax.fori_loop(0, NV // 2, pc, 0)

        def pm(v, carry):
            r = restore_v[pl.ds(v * LANES, LANES)]
            mask_v[pl.ds(v * LANES, LANES)] = jnp.where(
                r >= LEN_KEEP, jnp.float32(1.0), jnp.float32(0.0))
            return carry
        lax.fori_loop(0, NV, pm, 0)

        pltpu.sync_copy(restore_v, restore_hbm.at[row])
        pltpu.sync_copy(mask_v, mask_hbm.at[row])
        pltpu.sync_copy(shufg_v, shufg_hbm.at[row])


def _gather_body(x2d_hbm, prg_hbm, shufg_hbm,
                 xg_hbm, planes_hbm,
                 idxx_v, bufx0, bufx1, idxp_v, prgrow_v,
                 b0, b1, b2, b3, b4, b5, sem0, sem1):
    c = lax.axis_index("c")
    s = lax.axis_index("s")

    prow = c * ROWS_PER_SC + s // TPR
    seg = s % TPR

    for j in range(XC):
        pltpu.sync_copy(shufg_hbm.at[prow, pl.ds(seg * KSEG + j * CHUNK,
                                                 CHUNK)],
                        idxx_v.at[j])
    pltpu.sync_copy(shufg_hbm.at[prow, pl.ds(seg * SEG, SEG)], idxp_v)

    xbufs = [bufx0, bufx1]
    xsems = [sem0, sem1]
    descs = [None, None]
    descs[0] = pltpu.async_copy(x2d_hbm.at[idxx_v.at[0]], bufx0, sem0)

    pltpu.sync_copy(prg_hbm.at[prow], prgrow_v)
    planes = [b0, b1, b2, b3, b4, b5]

    def pg(i, carry):
        ids8 = (idxp_v[pl.ds(i * LANES, LANES)] & (L - 1)) * PRGW
        for col in range(6):
            planes[col][pl.ds(i * LANES, LANES)] = plsc.load_gather(
                prgrow_v, [ids8 + col])
        return carry
    lax.fori_loop(0, SEG // LANES, pg, 0)

    for j in range(XC):
        cur = j % 2
        if j + 1 < XC:
            nxt = (j + 1) % 2
            descs[nxt] = pltpu.async_copy(
                x2d_hbm.at[idxx_v.at[j + 1]], xbufs[nxt], xsems[nxt])
        descs[cur].wait()
        outbase = c * (ROWS_PER_SC * LEN_KEEP) + s * KSEG + j * CHUNK
        pltpu.sync_copy(xbufs[cur], xg_hbm.at[pl.ds(outbase, CHUNK)])

    @pl.when(seg == 0)
    def _keep_out():
        for col in range(6):
            base = col * (N * LEN_KEEP) + prow * LEN_KEEP
            pltpu.sync_copy(planes[col], planes_hbm.at[pl.ds(base, SEG)])

    @pl.when(seg != 0)
    def _mask_out():
        for col in range(6):
            base = (KEEPBLK + col * (N * LEN_MASK)
                    + prow * LEN_MASK + (seg - 1) * SEG)
            pltpu.sync_copy(planes[col], planes_hbm.at[pl.ds(base, SEG)])


@jax.jit
def _masker(noise, x2d, prg):
    i32 = jnp.int32
    f32 = jnp.float32
    mesh = plsc.VectorSubcoreMesh(
        core_axis_name="c", subcore_axis_name="s",
        num_cores=NC, num_subcores=NS)
    params = pltpu.CompilerParams(needs_layout_passes=False)

    sortk = pl.kernel(
        _sort_body,
        out_type=[
            jax.ShapeDtypeStruct((N, L), i32),
            jax.ShapeDtypeStruct((N, L), f32),
            jax.ShapeDtypeStruct((N, L), i32),
        ],
        mesh=mesh,
        compiler_params=params,
        scratch_types=[
            pltpu.VMEM((L,), f32),
            pltpu.VMEM((L,), i32),
            pltpu.VMEM((L,), i32),
            pltpu.VMEM((L,), i32),
            pltpu.VMEM((L,), i32),
            pltpu.VMEM((L,), i32),
            pltpu.VMEM((L,), i32),
            pltpu.VMEM((L,), i32),
            pltpu.VMEM((L,), i32),
            pltpu.VMEM((L,), f32),
        ],
    )
    ids_restore, mask, shufg = sortk(noise)

    gatherk = pl.kernel(
        _gather_body,
        out_type=[
            jax.ShapeDtypeStruct((N * LEN_KEEP, D), f32),
            jax.ShapeDtypeStruct((6 * N * L,), f32),
        ],
        mesh=mesh,
        compiler_params=params,
        scratch_types=[
            pltpu.VMEM((XC, CHUNK), i32),
            pltpu.VMEM((CHUNK, D), f32),
            pltpu.VMEM((CHUNK, D), f32),
            pltpu.VMEM((SEG,), i32),
            pltpu.VMEM((L * PRGW,), f32),
            pltpu.VMEM((SEG,), f32),
            pltpu.VMEM((SEG,), f32),
            pltpu.VMEM((SEG,), f32),
            pltpu.VMEM((SEG,), f32),
            pltpu.VMEM((SEG,), f32),
            pltpu.VMEM((SEG,), f32),
            pltpu.SemaphoreType.DMA,
            pltpu.SemaphoreType.DMA,
        ],
    )
    xg, planes = gatherk(x2d, prg, shufg)
    return xg, planes, ids_restore, mask


def kernel(x, points_xyz, rgb, noise):
    x2d = x.reshape(N * L, D)
    prg = jnp.concatenate(
        [points_xyz, rgb, jnp.zeros((N, L, PRGW - 6), jnp.float32)],
        axis=-1).reshape(N, L * PRGW)
    xg, planes, ids_restore, mask = _masker(noise, x2d, prg)
    nk = N * LEN_KEEP
    kp = planes[:6 * nk].reshape(6, N, LEN_KEEP)
    mp = planes[6 * nk:].reshape(6, N, LEN_MASK)
    pk = jnp.stack([kp[0], kp[1], kp[2]], axis=-1)
    rk = jnp.stack([kp[3], kp[4], kp[5]], axis=-1)
    pm = jnp.stack([mp[0], mp[1], mp[2]], axis=-1)
    rm = jnp.stack([mp[3], mp[4], mp[5]], axis=-1)
    return (xg.reshape(N, LEN_KEEP, D), mask, ids_restore, pk, pm, rk, rm)


# speedup vs baseline: 4.9352x; 1.0191x over previous
import jax
import jax.numpy as jnp
from jax import lax
from jax.experimental import pallas as pl
from jax.experimental.pallas import tpu as pltpu
from jax.experimental.pallas import tpu_sc as plsc

N, L, D = 8, 4096, 384
LEN_KEEP = 1024
LEN_MASK = L - LEN_KEEP
NC, NS, LANES = 2, 16, 16
NV = L // LANES
ROWS_PER_SC = N // NC
TPR = NS // ROWS_PER_SC
SEG = L // TPR
KSEG = LEN_KEEP // TPR
CHUNK = 64
XC = KSEG // CHUNK
PRGW = 8
KEEPBLK = 6 * N * LEN_KEEP


def _sort_body(noise_hbm, restore_hbm, mask_hbm, shufg_hbm,
               noise_v, ka, kb, pa, pb, hist, off, restore_v, shufg_v,
               mask_v):
    c = lax.axis_index("c")
    s = lax.axis_index("s")
    lane = lax.iota(jnp.int32, 16)
    zeros16 = jnp.zeros((LANES,), jnp.int32)
    ones16 = jnp.ones((LANES,), jnp.int32)

    @pl.when(s < ROWS_PER_SC)
    def _sort():
        row = c * ROWS_PER_SC + s
        pltpu.sync_copy(noise_hbm.at[row], noise_v)

        def p0(v, carry):
            f = noise_v[pl.ds(v * LANES, LANES)]
            k = lax.bitcast_convert_type(f, jnp.int32)
            e = v * LANES + lane
            slot = ((e & (NV - 1)) << 4) | (e >> 8)
            plsc.store_scatter(ka, [slot], k)
            plsc.store_scatter(pa, [slot], e)
            return carry
        lax.fori_loop(0, NV, p0, 0)

        bufs = [(ka, pa), (kb, pb)]
        for p in range(4):
            shift = 8 * p
            src_k, src_p = bufs[p % 2]
            dst_k, dst_p = bufs[(p + 1) % 2]
            last = p == 3
            ndig = 64 if last else 256

            def pz(h, carry):
                hist[pl.ds(h * LANES, LANES)] = zeros16
                return carry
            lax.fori_loop(0, ndig, pz, 0)

            def ph(v, carry):
                for u in range(2):
                    k = src_k[pl.ds((2 * v + u) * LANES, LANES)]
                    d = (k >> shift) & 0xFF
                    plsc.addupdate_scatter(hist, [(d << 4) | lane], ones16)
                return carry
            lax.fori_loop(0, NV // 2, ph, 0)

            def ps(h, carry):
                hv = hist[pl.ds(h * LANES, LANES)]
                inc = plsc.cumsum(hv)
                off[pl.ds(h * LANES, LANES)] = inc - hv + carry
                return carry + jnp.sum(hv)
            lax.fori_loop(0, ndig, ps, jnp.int32(0))

            def pc(v, carry):
                for u in range(2):
                    k = src_k[pl.ds((2 * v + u) * LANES, LANES)]
                    pay = src_p[pl.ds((2 * v + u) * LANES, LANES)]
                    d = (k >> shift) & 0xFF
                    hidx = (d << 4) | lane
                    pos = plsc.load_gather(off, [hidx])
                    plsc.store_scatter(off, [hidx], pos + 1)
                    if not last:
                        slot = ((pos & (NV - 1)) << 4) | (pos >> 8)
                        plsc.store_scatter(dst_k, [slot], k)
                        plsc.store_scatter(dst_p, [slot], pay)
                    else:
                        plsc.store_scatter(restore_v, [pay], pos)
                        plsc.store_scatter(shufg_v, [pos], pay + row * L)
                return carry
            lax.fori_loop(0, NV // 2, pc, 0)

        def pm(v, carry):
            r = restore_v[pl.ds(v * LANES, LANES)]
            mask_v[pl.ds(v * LANES, LANES)] = jnp.where(
                r >= LEN_KEEP, jnp.float32(1.0), jnp.float32(0.0))
            return carry
        lax.fori_loop(0, NV, pm, 0)

        pltpu.sync_copy(restore_v, restore_hbm.at[row])
        pltpu.sync_copy(mask_v, mask_hbm.at[row])
        pltpu.sync_copy(shufg_v, shufg_hbm.at[row])


def _gather_body(x2d_hbm, prg_hbm, shufg_hbm,
                 xg_hbm, planes_hbm,
                 idxx_v, bufx0, bufx1, idxp_v, prgrow_v,
                 b0, b1, b2, b3, b4, b5, sem0, sem1):
    c = lax.axis_index("c")
    s = lax.axis_index("s")

    prow = c * ROWS_PER_SC + s // TPR
    seg = s % TPR

    for j in range(XC):
        pltpu.sync_copy(shufg_hbm.at[prow, pl.ds(seg * KSEG + j * CHUNK,
                                                 CHUNK)],
                        idxx_v.at[j])
    pltpu.sync_copy(shufg_hbm.at[prow, pl.ds(seg * SEG, SEG)], idxp_v)

    xbufs = [bufx0, bufx1]
    xsems = [sem0, sem1]
    descs = [None, None]
    descs[0] = pltpu.async_copy(x2d_hbm.at[idxx_v.at[0]], bufx0, sem0)

    pltpu.sync_copy(prg_hbm.at[prow], prgrow_v)
    planes = [b0, b1, b2, b3, b4, b5]

    def pg(i, carry):
        ids8 = (idxp_v[pl.ds(i * LANES, LANES)] & (L - 1)) * PRGW
        for col in range(6):
            planes[col][pl.ds(i * LANES, LANES)] = plsc.load_gather(
                prgrow_v, [ids8 + col])
        return carry
    lax.fori_loop(0, SEG // LANES, pg, 0)

    for j in range(XC):
        cur = j % 2
        if j + 1 < XC:
            nxt = (j + 1) % 2
            descs[nxt] = pltpu.async_copy(
                x2d_hbm.at[idxx_v.at[j + 1]], xbufs[nxt], xsems[nxt])
        descs[cur].wait()
        outbase = c * (ROWS_PER_SC * LEN_KEEP) + s * KSEG + j * CHUNK
        pltpu.sync_copy(xbufs[cur], xg_hbm.at[pl.ds(outbase, CHUNK)])

    @pl.when(seg == 0)
    def _keep_out():
        for col in range(6):
            base = col * (N * LEN_KEEP) + prow * LEN_KEEP
            pltpu.sync_copy(planes[col], planes_hbm.at[pl.ds(base, SEG)])

    @pl.when(seg != 0)
    def _mask_out():
        for col in range(6):
            base = (KEEPBLK + col * (N * LEN_MASK)
                    + prow * LEN_MASK + (seg - 1) * SEG)
            pltpu.sync_copy(planes[col], planes_hbm.at[pl.ds(base, SEG)])


@jax.jit
def _masker(noise, x2d, prg):
    i32 = jnp.int32
    f32 = jnp.float32
    mesh = plsc.VectorSubcoreMesh(
        core_axis_name="c", subcore_axis_name="s",
        num_cores=NC, num_subcores=NS)
    params = pltpu.CompilerParams(needs_layout_passes=False)

    sortk = pl.kernel(
        _sort_body,
        out_type=[
            jax.ShapeDtypeStruct((N, L), i32),
            jax.ShapeDtypeStruct((N, L), f32),
            jax.ShapeDtypeStruct((N, L), i32),
        ],
        mesh=mesh,
        compiler_params=params,
        scratch_types=[
            pltpu.VMEM((L,), f32),
            pltpu.VMEM((L,), i32),
            pltpu.VMEM((L,), i32),
            pltpu.VMEM((L,), i32),
            pltpu.VMEM((L,), i32),
            pltpu.VMEM((L,), i32),
            pltpu.VMEM((L,), i32),
            pltpu.VMEM((L,), i32),
            pltpu.VMEM((L,), i32),
            pltpu.VMEM((L,), f32),
        ],
    )
    ids_restore, mask, shufg = sortk(noise)

    gatherk = pl.kernel(
        _gather_body,
        out_type=[
            jax.ShapeDtypeStruct((N * LEN_KEEP, D), f32),
            jax.ShapeDtypeStruct((6 * N * L,), f32),
        ],
        mesh=mesh,
        compiler_params=params,
        scratch_types=[
            pltpu.VMEM((XC, CHUNK), i32),
            pltpu.VMEM((CHUNK, D), f32),
            pltpu.VMEM((CHUNK, D), f32),
            pltpu.VMEM((SEG,), i32),
            pltpu.VMEM((L * PRGW,), f32),
            pltpu.VMEM((SEG,), f32),
            pltpu.VMEM((SEG,), f32),
            pltpu.VMEM((SEG,), f32),
            pltpu.VMEM((SEG,), f32),
            pltpu.VMEM((SEG,), f32),
            pltpu.VMEM((SEG,), f32),
            pltpu.SemaphoreType.DMA,
            pltpu.SemaphoreType.DMA,
        ],
    )
    xg, planes = gatherk(x2d, prg, shufg)
    return xg, planes, ids_restore, mask


def kernel(x, points_xyz, rgb, noise):
    x2d = x.reshape(N * L, D)
    prg = jnp.concatenate(
        [points_xyz, rgb, jnp.zeros((N, L, PRGW - 6), jnp.float32)],
        axis=-1).reshape(N, L * PRGW)
    xg, planes, ids_restore, mask = _masker(noise, x2d, prg)
    nk = N * LEN_KEEP
    kp = planes[:6 * nk].reshape(6, N, LEN_KEEP)
    mp = planes[6 * nk:].reshape(6, N, LEN_MASK)
    pk = jnp.stack([kp[0], kp[1], kp[2]], axis=-1)
    rk = jnp.stack([kp[3], kp[4], kp[5]], axis=-1)
    pm = jnp.stack([mp[0], mp[1], mp[2]], axis=-1)
    rm = jnp.stack([mp[3], mp[4], mp[5]], axis=-1)
    return (xg.reshape(N, LEN_KEEP, D), mask, ids_restore, pk, pm, rk, rm)
